# baseline (speedup 1.0000x reference)
import functools
import math

import jax
import jax.numpy as jnp
from jax import lax
from jax.experimental import pallas as pl
from jax.experimental.pallas import tpu as pltpu
from jax.experimental.pallas import tpu_sc as plsc

D = 64
DP = 128
BATCH = 4096
SEQ = 200
VOCAB = 1000000
B_TOT = BATCH * SEQ
NC, NS = 2, 16
NW = NC * NS
IDX_PER_W = B_TOT // NW
CHUNK = 160
NCHUNKS = IDX_PER_W // CHUNK
SPLIT = 128
REM = CHUNK - SPLIT
NBUF = 2
SCALE = math.sqrt(D)
LANES = 16

_mesh = plsc.VectorSubcoreMesh(core_axis_name="c", subcore_axis_name="s")


@functools.partial(
    pl.kernel,
    mesh=_mesh,
    out_type=jax.ShapeDtypeStruct((B_TOT, DP), jnp.float32),
    compiler_params=pltpu.CompilerParams(use_tc_tiling_on_sc=True),
    scratch_types=[
        pltpu.VMEM((IDX_PER_W,), jnp.int32),
        pltpu.VMEM((CHUNK, DP), jnp.float32),
        pltpu.VMEM((CHUNK, DP), jnp.float32),
        pltpu.VMEM((CHUNK, DP), jnp.float32),
        pltpu.VMEM((CHUNK, DP), jnp.float32),
        pltpu.SemaphoreType.DMA,
        pltpu.SemaphoreType.DMA,
        pltpu.SemaphoreType.DMA,
        pltpu.SemaphoreType.DMA,
    ],
)
def _emb(x_hbm, w_hbm, out_hbm, idx_v, ga, gb, sa, sb, gs0, gs1, os0, os1):
    wid = lax.axis_index("s") * NC + lax.axis_index("c")
    base = wid * IDX_PER_W
    gbuf = [ga, gb]
    sbuf = [sa, sb]
    gsem = [gs0, gs1]
    osem = [os0, os1]

    pltpu.sync_copy(x_hbm.at[pl.ds(base, IDX_PER_W)], idx_v)

    def start_gather(g, b):
        pltpu.async_copy(w_hbm.at[idx_v.at[pl.ds(g * CHUNK, SPLIT)]],
                         gbuf[b].at[pl.ds(0, SPLIT)], gsem[b])
        pltpu.async_copy(w_hbm.at[idx_v.at[pl.ds(g * CHUNK + SPLIT, REM)]],
                         gbuf[b].at[pl.ds(SPLIT, REM)], gsem[b])

    def wait_gather(b):
        pltpu.make_async_copy(w_hbm.at[pl.ds(0, CHUNK)], gbuf[b],
                              gsem[b]).wait()

    def scale(b):
        @plsc.parallel_loop(0, CHUNK, 1, unroll=4)
        def _(c):
            for q in range(D // LANES):
                sl = pl.ds(q * LANES, LANES)
                sbuf[b][c, sl] = gbuf[b][c, sl] * SCALE

    def start_scatter(g, b):
        pltpu.async_copy(sbuf[b], out_hbm.at[pl.ds(base + g * CHUNK, CHUNK)],
                         osem[b])

    def wait_scatter(b):
        pltpu.make_async_copy(sbuf[b], out_hbm.at[pl.ds(base, CHUNK)],
                              osem[b]).wait()

    def do_chunk(g, b, first, last):
        wait_gather(b)
        if not first:
            wait_scatter(b)
        scale(b)
        start_scatter(g, b)
        if not last:
            start_gather(g + NBUF, b)

    for b in range(NBUF):
        start_gather(b, b)
    for b in range(NBUF):
        do_chunk(b, b, True, False)

    def pair_body(p, carry):
        for b in range(NBUF):
            do_chunk(p * NBUF + b, b, False, False)
        return carry

    lax.fori_loop(1, NCHUNKS // NBUF - 1, pair_body, 0)

    for b in range(NBUF):
        do_chunk(NCHUNKS - NBUF + b, b, False, True)
    for b in range(NBUF):
        wait_scatter(b)


BK = 16384
NBK = -(-VOCAB // BK)


def _prep_body(wt_ref, o_ref):
    o_ref[:, :D] = wt_ref[...].T


_prep = pl.pallas_call(
    _prep_body,
    grid=(NBK,),
    in_specs=[pl.BlockSpec((D, BK), lambda i: (0, i))],
    out_specs=pl.BlockSpec((BK, DP), lambda i: (i, 0)),
    out_shape=jax.ShapeDtypeStruct((VOCAB, DP), jnp.float32),
)


def kernel(x, weight):
    w128 = _prep(weight.T)
    out = _emb(x.reshape(B_TOT), w128)
    return out[:, :D].reshape(BATCH, SEQ, D)


# speedup vs baseline: 1.0016x; 1.0016x over previous
import functools
import math

import jax
import jax.numpy as jnp
from jax import lax
from jax.experimental import pallas as pl
from jax.experimental.pallas import tpu as pltpu
from jax.experimental.pallas import tpu_sc as plsc

D = 64
DP = 128
BATCH = 4096
SEQ = 200
VOCAB = 1000000
B_TOT = BATCH * SEQ
NC, NS = 2, 16
NW = NC * NS
IDX_PER_W = B_TOT // NW
CHUNK = 160
NCHUNKS = IDX_PER_W // CHUNK
SPLIT = 128
REM = CHUNK - SPLIT
NBUF = 2
SCALE = math.sqrt(D)
LANES = 16

_mesh = plsc.VectorSubcoreMesh(core_axis_name="c", subcore_axis_name="s")


@functools.partial(
    pl.kernel,
    mesh=_mesh,
    out_type=jax.ShapeDtypeStruct((B_TOT, DP), jnp.float32),
    compiler_params=pltpu.CompilerParams(use_tc_tiling_on_sc=True),
    scratch_types=[
        pltpu.VMEM((IDX_PER_W,), jnp.int32),
        pltpu.VMEM((CHUNK, DP), jnp.float32),
        pltpu.VMEM((CHUNK, DP), jnp.float32),
        pltpu.VMEM((CHUNK, DP), jnp.float32),
        pltpu.VMEM((CHUNK, DP), jnp.float32),
        pltpu.SemaphoreType.DMA,
        pltpu.SemaphoreType.DMA,
        pltpu.SemaphoreType.DMA,
        pltpu.SemaphoreType.DMA,
    ],
)
def _emb(x_hbm, w_hbm, out_hbm, idx_v, ga, gb, sa, sb, gs0, gs1, os0, os1):
    wid = lax.axis_index("s") * NC + lax.axis_index("c")
    base = wid * IDX_PER_W
    gbuf = [ga, gb]
    sbuf = [sa, sb]
    gsem = [gs0, gs1]
    osem = [os0, os1]

    pltpu.sync_copy(x_hbm.at[pl.ds(base, IDX_PER_W)], idx_v)

    def start_gather(g, b):
        pltpu.async_copy(w_hbm.at[idx_v.at[pl.ds(g * CHUNK, CHUNK)]],
                         gbuf[b], gsem[b])

    def wait_gather(b):
        pltpu.make_async_copy(w_hbm.at[pl.ds(0, CHUNK)], gbuf[b],
                              gsem[b]).wait()

    def scale(b):
        @plsc.parallel_loop(0, CHUNK, 1, unroll=4)
        def _(c):
            for q in range(D // LANES):
                sl = pl.ds(q * LANES, LANES)
                sbuf[b][c, sl] = gbuf[b][c, sl] * SCALE

    def start_scatter(g, b):
        pltpu.async_copy(sbuf[b], out_hbm.at[pl.ds(base + g * CHUNK, CHUNK)],
                         osem[b])

    def wait_scatter(b):
        pltpu.make_async_copy(sbuf[b], out_hbm.at[pl.ds(base, CHUNK)],
                              osem[b]).wait()

    def do_chunk(g, b, first, last):
        wait_gather(b)
        if not first:
            wait_scatter(b)
        scale(b)
        start_scatter(g, b)
        if not last:
            start_gather(g + NBUF, b)

    for b in range(NBUF):
        start_gather(b, b)
    for b in range(NBUF):
        do_chunk(b, b, True, False)

    def pair_body(p, carry):
        for b in range(NBUF):
            do_chunk(p * NBUF + b, b, False, False)
        return carry

    lax.fori_loop(1, NCHUNKS // NBUF - 1, pair_body, 0)

    for b in range(NBUF):
        do_chunk(NCHUNKS - NBUF + b, b, False, True)
    for b in range(NBUF):
        wait_scatter(b)


BK = 16384
NBK = -(-VOCAB // BK)


def _prep_body(wt_ref, o_ref):
    o_ref[:, :D] = wt_ref[...].T


_prep = pl.pallas_call(
    _prep_body,
    grid=(NBK,),
    in_specs=[pl.BlockSpec((D, BK), lambda i: (0, i))],
    out_specs=pl.BlockSpec((BK, DP), lambda i: (i, 0)),
    out_shape=jax.ShapeDtypeStruct((VOCAB, DP), jnp.float32),
)


def kernel(x, weight):
    w128 = _prep(weight.T)
    out = _emb(x.reshape(B_TOT), w128)
    return out[:, :D].reshape(BATCH, SEQ, D)


# speedup vs baseline: 1.0204x; 1.0188x over previous
import functools
import math

import jax
import jax.numpy as jnp
from jax import lax
from jax.experimental import pallas as pl
from jax.experimental.pallas import tpu as pltpu
from jax.experimental.pallas import tpu_sc as plsc

D = 64
DP = 128
BATCH = 4096
SEQ = 200
VOCAB = 1000000
NC, NS = 2, 16
NW = NC * NS
ROWS_PER_W = BATCH // NW
SPLIT = 128
REM = SEQ - SPLIT
NBUF = 2
SCALE = math.sqrt(D)
LANES = 16

_mesh = plsc.VectorSubcoreMesh(core_axis_name="c", subcore_axis_name="s")


@functools.partial(
    pl.kernel,
    mesh=_mesh,
    out_type=jax.ShapeDtypeStruct((BATCH, SEQ, DP), jnp.float32),
    compiler_params=pltpu.CompilerParams(use_tc_tiling_on_sc=True),
    scratch_types=[
        pltpu.VMEM((ROWS_PER_W, SEQ), jnp.int32),
        pltpu.VMEM((SEQ, DP), jnp.float32),
        pltpu.VMEM((SEQ, DP), jnp.float32),
        pltpu.SemaphoreType.DMA,
        pltpu.SemaphoreType.DMA,
        pltpu.SemaphoreType.DMA,
        pltpu.SemaphoreType.DMA,
    ],
)
def _emb(x_hbm, w_hbm, out_hbm, idx_v, rows0, rows1, gs0, gs1, os0, os1):
    wid = lax.axis_index("s") * NC + lax.axis_index("c")
    xr0 = wid * ROWS_PER_W
    rows = [rows0, rows1]
    gsem = [gs0, gs1]
    osem = [os0, os1]

    pltpu.sync_copy(x_hbm.at[pl.ds(xr0, ROWS_PER_W)], idx_v)

    def start_gather(g, b):
        pltpu.async_copy(
            w_hbm.at[idx_v.at[g, pl.ds(0, SPLIT)]],
            rows[b].at[pl.ds(0, SPLIT)],
            gsem[b],
        )
        pltpu.async_copy(
            w_hbm.at[idx_v.at[g, pl.ds(SPLIT, REM)]],
            rows[b].at[pl.ds(SPLIT, REM)],
            gsem[b],
        )

    def wait_gather(b):
        pltpu.make_async_copy(w_hbm.at[pl.ds(0, SEQ)], rows[b], gsem[b]).wait()

    def scale(b):
        @plsc.parallel_loop(0, SEQ, 1, unroll=4)
        def _(c):
            for q in range(D // LANES):
                sl = pl.ds(q * LANES, LANES)
                rows[b][c, sl] = rows[b][c, sl] * SCALE

    def start_scatter(g, b):
        pltpu.async_copy(rows[b], out_hbm.at[xr0 + g], osem[b])

    def wait_scatter(b):
        pltpu.make_async_copy(rows[b], out_hbm.at[xr0], osem[b]).wait()

    for b in range(NBUF):
        start_gather(b, b)

    def pair_body(p, carry):
        for b in range(NBUF):
            g = p * NBUF + b
            wait_gather(b)
            scale(b)
            start_scatter(g, b)
            wait_scatter(b)
            start_gather(g + NBUF, b)
        return carry

    lax.fori_loop(0, ROWS_PER_W // NBUF - 1, pair_body, 0)

    for b in range(NBUF):
        g = ROWS_PER_W - NBUF + b
        wait_gather(b)
        scale(b)
        start_scatter(g, b)
        wait_scatter(b)


BK = 32768
NBK = -(-VOCAB // BK)


def _prep_body(wt_ref, o_ref):
    o_ref[:, :D] = wt_ref[...].T


_prep = pl.pallas_call(
    _prep_body,
    grid=(NBK,),
    in_specs=[pl.BlockSpec((D, BK), lambda i: (0, i))],
    out_specs=pl.BlockSpec((BK, DP), lambda i: (i, 0)),
    out_shape=jax.ShapeDtypeStruct((VOCAB, DP), jnp.float32),
)


def kernel(x, weight):
    w128 = _prep(weight.T)
    w128 = jax.lax.optimization_barrier(w128)
    return _emb(x, w128)[:, :, :D]
